# baseline (speedup 1.0000x reference)
import functools

import jax
import jax.numpy as jnp
from jax import lax
from jax.experimental import pallas as pl
from jax.experimental.pallas import tpu as pltpu
from jax.experimental.pallas import tpu_sc as plsc

N = 10000
NPAD = 10240
D = 128
H = 64
G = 64
OUT = 6
E = 320000
NW = 32
CHUNK = 128
CF = 126
CS = 34
EPAD = 16 * (CF + CS) * CHUNK
ROWS_PER_TILE = NPAD // 16
BLK = 1024
NB = NPAD // BLK


def _segment_sum_sc(u, src_f, dst_f, src_s, dst_s, zrows):
    mesh = plsc.VectorSubcoreMesh(core_axis_name="c", subcore_axis_name="s")

    @functools.partial(
        pl.kernel,
        out_type=jax.ShapeDtypeStruct((2 * NPAD, H), jnp.float32),
        mesh=mesh,
        compiler_params=pltpu.CompilerParams(use_tc_tiling_on_sc=False),
        scratch_types=[
            pltpu.VMEM((CF, CHUNK), jnp.int32),
            pltpu.VMEM((CF, CHUNK), jnp.int32),
            pltpu.VMEM((CHUNK, H), jnp.float32),
            pltpu.VMEM((CHUNK, H), jnp.float32),
            pltpu.VMEM_SHARED((NPAD, H), jnp.float32),
            pltpu.SemaphoreType.DMA,
            pltpu.SemaphoreType.DMA,
        ],
    )
    def body(u_hbm, srcf_hbm, dstf_hbm, srcs_hbm, dsts_hbm, z_hbm, out_hbm,
             src_v, dst_v, rows0, rows1, acc, sem0, sem1):
        cid = lax.axis_index("c")
        sid = lax.axis_index("s")
        is_fast = cid == 0

        @pl.when(is_fast)
        def _():
            pltpu.sync_copy(srcf_hbm.at[sid], src_v)
            pltpu.sync_copy(dstf_hbm.at[sid], dst_v)

        @pl.when(jnp.logical_not(is_fast))
        def _():
            pltpu.sync_copy(srcs_hbm.at[sid], src_v.at[pl.ds(0, CS)])
            pltpu.sync_copy(dsts_hbm.at[sid], dst_v.at[pl.ds(0, CS)])

        pltpu.sync_copy(z_hbm, acc.at[pl.ds(sid * ROWS_PER_TILE, ROWS_PER_TILE)])
        plsc.subcore_barrier()

        cnt = jnp.where(is_fast, CF, CS)

        pltpu.async_copy(u_hbm.at[src_v.at[0]], rows0, sem0)

        def step(i, carry):
            j = 2 * i
            pltpu.async_copy(u_hbm.at[src_v.at[j + 1]], rows1, sem1)
            pltpu.make_async_copy(u_hbm.at[src_v.at[j]], rows0, sem0).wait()
            pltpu.sync_copy(rows0, acc.at[dst_v.at[j]], add=True)

            @pl.when(j + 2 < cnt)
            def _():
                pltpu.async_copy(u_hbm.at[src_v.at[j + 2]], rows0, sem0)

            pltpu.make_async_copy(u_hbm.at[src_v.at[j + 1]], rows1, sem1).wait()
            pltpu.sync_copy(rows1, acc.at[dst_v.at[j + 1]], add=True)
            return carry

        lax.fori_loop(0, cnt // 2, step, 0)
        plsc.subcore_barrier()
        pltpu.sync_copy(
            acc.at[pl.ds(sid * ROWS_PER_TILE, ROWS_PER_TILE)],
            out_hbm.at[pl.ds(cid * NPAD + sid * ROWS_PER_TILE, ROWS_PER_TILE)])

    return body(u, src_f, dst_f, src_s, dst_s, zrows)


def _k1(xp, W1a):
    def body(x_ref, w_ref, o_ref):
        o_ref[...] = jnp.dot(x_ref[...], w_ref[...],
                             preferred_element_type=jnp.float32)

    return pl.pallas_call(
        body,
        grid=(NB,),
        in_specs=[pl.BlockSpec((BLK, D), lambda i: (i, 0)),
                  pl.BlockSpec((D, H), lambda i: (0, 0))],
        out_specs=pl.BlockSpec((BLK, H), lambda i: (i, 0)),
        out_shape=jax.ShapeDtypeStruct((NPAD, H), jnp.float32),
    )(xp, W1a)


def _k2(p, u, b1, W2, b2, Wn):
    def body(p_ref, u_ref, b1_ref, w2_ref, b2_ref, wn_ref, o_ref):
        t = jnp.maximum(p_ref[0] + p_ref[1] + u_ref[...] + b1_ref[...], 0.0)
        h = jnp.maximum(
            jnp.dot(t, w2_ref[...], preferred_element_type=jnp.float32)
            + b2_ref[...], 0.0)
        o_ref[...] = jnp.dot(h, wn_ref[...], preferred_element_type=jnp.float32)

    return pl.pallas_call(
        body,
        grid=(NB,),
        in_specs=[pl.BlockSpec((2, BLK, H), lambda i: (0, i, 0)),
                  pl.BlockSpec((BLK, H), lambda i: (i, 0)),
                  pl.BlockSpec((1, H), lambda i: (0, 0)),
                  pl.BlockSpec((H, H), lambda i: (0, 0)),
                  pl.BlockSpec((1, H), lambda i: (0, 0)),
                  pl.BlockSpec((H, H), lambda i: (0, 0))],
        out_specs=pl.BlockSpec((BLK, H), lambda i: (i, 0)),
        out_shape=jax.ShapeDtypeStruct((NPAD, H), jnp.float32),
    )(p, u, b1, W2, b2, Wn)


def _k3(q, v, b1, W2, b2, batch3, Wf1, bf1, Wf2, bf2):
    def body(q_ref, v_ref, b1_ref, w2_ref, b2_ref, bt_ref, wf1_ref, bf1_ref,
             wf2_ref, bf2_ref, o_ref, sum_ref, cnt_ref):
        i = pl.program_id(0)
        t = jnp.maximum(q_ref[0] + q_ref[1] + v_ref[...] + b1_ref[...], 0.0)
        h2 = jnp.maximum(
            jnp.dot(t, w2_ref[...], preferred_element_type=jnp.float32)
            + b2_ref[...], 0.0)
        bv = bt_ref[0]
        oh = (lax.broadcasted_iota(jnp.int32, (G, BLK), 0)
              == jnp.broadcast_to(bv, (G, BLK))).astype(jnp.float32)
        s = lax.dot_general(oh, h2, (((1,), (0,)), ((), ())),
                            preferred_element_type=jnp.float32)
        c = lax.dot_general(oh, jnp.ones((BLK, H), jnp.float32),
                            (((1,), (0,)), ((), ())),
                            preferred_element_type=jnp.float32)

        @pl.when(i == 0)
        def _():
            sum_ref[...] = s
            cnt_ref[...] = c

        @pl.when(i > 0)
        def _():
            sum_ref[...] += s
            cnt_ref[...] += c

        @pl.when(i == NB - 1)
        def _():
            pooled = sum_ref[...] / jnp.maximum(cnt_ref[...], 1.0)
            o1 = jnp.maximum(
                jnp.dot(pooled, wf1_ref[...], preferred_element_type=jnp.float32)
                + bf1_ref[...], 0.0)
            logits = (jnp.dot(o1, wf2_ref[...], preferred_element_type=jnp.float32)
                      + bf2_ref[...])
            o_ref[...] = 1.0 / (1.0 + jnp.exp(-logits))

    return pl.pallas_call(
        body,
        grid=(NB,),
        in_specs=[pl.BlockSpec((2, BLK, H), lambda i: (0, i, 0)),
                  pl.BlockSpec((BLK, H), lambda i: (i, 0)),
                  pl.BlockSpec((1, H), lambda i: (0, 0)),
                  pl.BlockSpec((H, H), lambda i: (0, 0)),
                  pl.BlockSpec((1, H), lambda i: (0, 0)),
                  pl.BlockSpec((1, 1, BLK), lambda i: (i, 0, 0)),
                  pl.BlockSpec((H, H), lambda i: (0, 0)),
                  pl.BlockSpec((1, H), lambda i: (0, 0)),
                  pl.BlockSpec((H, OUT), lambda i: (0, 0)),
                  pl.BlockSpec((1, OUT), lambda i: (0, 0))],
        out_specs=pl.BlockSpec((G, OUT), lambda i: (0, 0)),
        out_shape=jax.ShapeDtypeStruct((G, OUT), jnp.float32),
        scratch_shapes=[pltpu.VMEM((G, H), jnp.float32),
                        pltpu.VMEM((G, H), jnp.float32)],
    )(q, v, b1, W2, b2, batch3, Wf1, bf1, Wf2, bf2)


def kernel(x, edge_index, batch, W1a, b1a, W2a, b2a, W1b, b1b, W2b, b2b,
           Wf1, bf1, Wf2, bf2):
    xp = jnp.pad(x, ((0, NPAD - N), (0, 0)))
    src = edge_index[0]
    dst = edge_index[1]
    fill = jnp.full((EPAD - E,), N, jnp.int32)
    nf = 16 * CF * CHUNK
    srcp = jnp.concatenate([src, fill])
    dstp = jnp.concatenate([dst, fill])
    src_f = srcp[:nf].reshape(16, CF, CHUNK)
    dst_f = dstp[:nf].reshape(16, CF, CHUNK)
    src_s = srcp[nf:].reshape(16, CS, CHUNK)
    dst_s = dstp[nf:].reshape(16, CS, CHUNK)
    batch3 = jnp.concatenate(
        [batch, jnp.full((NPAD - N,), G, jnp.int32)]).reshape(NB, 1, BLK)

    zrows = jnp.zeros((ROWS_PER_TILE, H), jnp.float32)
    u = _k1(xp, W1a)
    p = _segment_sum_sc(u, src_f, dst_f, src_s, dst_s, zrows).reshape(2, NPAD, H)
    v = _k2(p, u, b1a.reshape(1, H), W2a, b2a.reshape(1, H), W1b)
    q = _segment_sum_sc(v, src_f, dst_f, src_s, dst_s, zrows).reshape(2, NPAD, H)
    out = _k3(q, v, b1b.reshape(1, H), W2b, b2b.reshape(1, H), batch3,
              Wf1, bf1.reshape(1, H), Wf2, bf2.reshape(1, OUT))
    return out


# speedup vs baseline: 2.0534x; 2.0534x over previous
import functools

import jax
import jax.numpy as jnp
from jax import lax
from jax.experimental import pallas as pl
from jax.experimental.pallas import tpu as pltpu
from jax.experimental.pallas import tpu_sc as plsc

N = 10000
NPAD = 10240
D = 128
H = 64
G = 64
OUT = 6
E = 320000
NW = 32
CHUNK = 128
CF = 80
CS = 80
EPAD = 16 * (CF + CS) * CHUNK
ROWS_PER_TILE = NPAD // 16
BLK = 1024
NB = NPAD // BLK


def _segment_sum_sc(u, src_f, dst_f, src_s, dst_s, zrows):
    mesh = plsc.VectorSubcoreMesh(core_axis_name="c", subcore_axis_name="s")

    @functools.partial(
        pl.kernel,
        out_type=jax.ShapeDtypeStruct((2 * NPAD, H), jnp.float32),
        mesh=mesh,
        compiler_params=pltpu.CompilerParams(use_tc_tiling_on_sc=False),
        scratch_types=[
            pltpu.VMEM((CF, CHUNK), jnp.int32),
            pltpu.VMEM((CF, CHUNK), jnp.int32),
            pltpu.VMEM((CHUNK, H), jnp.float32),
            pltpu.VMEM((CHUNK, H), jnp.float32),
            pltpu.VMEM_SHARED((NPAD, H), jnp.float32),
            pltpu.VMEM_SHARED((NPAD, H), jnp.float32),
            pltpu.SemaphoreType.DMA,
            pltpu.SemaphoreType.DMA,
        ],
    )
    def body(u_hbm, srcf_hbm, dstf_hbm, srcs_hbm, dsts_hbm, z_hbm, out_hbm,
             src_v, dst_v, rows0, rows1, acc, u_s, sem0, sem1):
        cid = lax.axis_index("c")
        sid = lax.axis_index("s")
        is_fast = cid == 0

        @pl.when(is_fast)
        def _():
            pltpu.sync_copy(srcf_hbm.at[sid], src_v)
            pltpu.sync_copy(dstf_hbm.at[sid], dst_v)

        @pl.when(jnp.logical_not(is_fast))
        def _():
            pltpu.sync_copy(srcs_hbm.at[sid], src_v.at[pl.ds(0, CS)])
            pltpu.sync_copy(dsts_hbm.at[sid], dst_v.at[pl.ds(0, CS)])

        pltpu.sync_copy(
            u_hbm.at[pl.ds(sid * ROWS_PER_TILE, ROWS_PER_TILE)],
            u_s.at[pl.ds(sid * ROWS_PER_TILE, ROWS_PER_TILE)])
        pltpu.sync_copy(z_hbm, acc.at[pl.ds(sid * ROWS_PER_TILE, ROWS_PER_TILE)])
        plsc.subcore_barrier()

        cnt = jnp.where(is_fast, CF, CS)

        pltpu.async_copy(u_s.at[src_v.at[0]], rows0, sem0)

        def step(i, carry):
            j = 2 * i
            pltpu.async_copy(u_s.at[src_v.at[j + 1]], rows1, sem1)
            pltpu.make_async_copy(u_s.at[src_v.at[j]], rows0, sem0).wait()
            pltpu.sync_copy(rows0, acc.at[dst_v.at[j]], add=True)

            @pl.when(j + 2 < cnt)
            def _():
                pltpu.async_copy(u_s.at[src_v.at[j + 2]], rows0, sem0)

            pltpu.make_async_copy(u_s.at[src_v.at[j + 1]], rows1, sem1).wait()
            pltpu.sync_copy(rows1, acc.at[dst_v.at[j + 1]], add=True)
            return carry

        lax.fori_loop(0, cnt // 2, step, 0)
        plsc.subcore_barrier()
        pltpu.sync_copy(
            acc.at[pl.ds(sid * ROWS_PER_TILE, ROWS_PER_TILE)],
            out_hbm.at[pl.ds(cid * NPAD + sid * ROWS_PER_TILE, ROWS_PER_TILE)])

    return body(u, src_f, dst_f, src_s, dst_s, zrows)


def _k1(xp, W1a):
    def body(x_ref, w_ref, o_ref):
        o_ref[...] = jnp.dot(x_ref[...], w_ref[...],
                             preferred_element_type=jnp.float32)

    return pl.pallas_call(
        body,
        grid=(NB,),
        in_specs=[pl.BlockSpec((BLK, D), lambda i: (i, 0)),
                  pl.BlockSpec((D, H), lambda i: (0, 0))],
        out_specs=pl.BlockSpec((BLK, H), lambda i: (i, 0)),
        out_shape=jax.ShapeDtypeStruct((NPAD, H), jnp.float32),
    )(xp, W1a)


def _k2(p, u, b1, W2, b2, Wn):
    def body(p_ref, u_ref, b1_ref, w2_ref, b2_ref, wn_ref, o_ref):
        t = jnp.maximum(p_ref[0] + p_ref[1] + u_ref[...] + b1_ref[...], 0.0)
        h = jnp.maximum(
            jnp.dot(t, w2_ref[...], preferred_element_type=jnp.float32)
            + b2_ref[...], 0.0)
        o_ref[...] = jnp.dot(h, wn_ref[...], preferred_element_type=jnp.float32)

    return pl.pallas_call(
        body,
        grid=(NB,),
        in_specs=[pl.BlockSpec((2, BLK, H), lambda i: (0, i, 0)),
                  pl.BlockSpec((BLK, H), lambda i: (i, 0)),
                  pl.BlockSpec((1, H), lambda i: (0, 0)),
                  pl.BlockSpec((H, H), lambda i: (0, 0)),
                  pl.BlockSpec((1, H), lambda i: (0, 0)),
                  pl.BlockSpec((H, H), lambda i: (0, 0))],
        out_specs=pl.BlockSpec((BLK, H), lambda i: (i, 0)),
        out_shape=jax.ShapeDtypeStruct((NPAD, H), jnp.float32),
    )(p, u, b1, W2, b2, Wn)


def _k3(q, v, b1, W2, b2, batch3, Wf1, bf1, Wf2, bf2):
    def body(q_ref, v_ref, b1_ref, w2_ref, b2_ref, bt_ref, wf1_ref, bf1_ref,
             wf2_ref, bf2_ref, o_ref, sum_ref, cnt_ref):
        i = pl.program_id(0)
        t = jnp.maximum(q_ref[0] + q_ref[1] + v_ref[...] + b1_ref[...], 0.0)
        h2 = jnp.maximum(
            jnp.dot(t, w2_ref[...], preferred_element_type=jnp.float32)
            + b2_ref[...], 0.0)
        bv = bt_ref[0]
        oh = (lax.broadcasted_iota(jnp.int32, (G, BLK), 0)
              == jnp.broadcast_to(bv, (G, BLK))).astype(jnp.float32)
        s = lax.dot_general(oh, h2, (((1,), (0,)), ((), ())),
                            preferred_element_type=jnp.float32)
        c = lax.dot_general(oh, jnp.ones((BLK, H), jnp.float32),
                            (((1,), (0,)), ((), ())),
                            preferred_element_type=jnp.float32)

        @pl.when(i == 0)
        def _():
            sum_ref[...] = s
            cnt_ref[...] = c

        @pl.when(i > 0)
        def _():
            sum_ref[...] += s
            cnt_ref[...] += c

        @pl.when(i == NB - 1)
        def _():
            pooled = sum_ref[...] / jnp.maximum(cnt_ref[...], 1.0)
            o1 = jnp.maximum(
                jnp.dot(pooled, wf1_ref[...], preferred_element_type=jnp.float32)
                + bf1_ref[...], 0.0)
            logits = (jnp.dot(o1, wf2_ref[...], preferred_element_type=jnp.float32)
                      + bf2_ref[...])
            o_ref[...] = 1.0 / (1.0 + jnp.exp(-logits))

    return pl.pallas_call(
        body,
        grid=(NB,),
        in_specs=[pl.BlockSpec((2, BLK, H), lambda i: (0, i, 0)),
                  pl.BlockSpec((BLK, H), lambda i: (i, 0)),
                  pl.BlockSpec((1, H), lambda i: (0, 0)),
                  pl.BlockSpec((H, H), lambda i: (0, 0)),
                  pl.BlockSpec((1, H), lambda i: (0, 0)),
                  pl.BlockSpec((1, 1, BLK), lambda i: (i, 0, 0)),
                  pl.BlockSpec((H, H), lambda i: (0, 0)),
                  pl.BlockSpec((1, H), lambda i: (0, 0)),
                  pl.BlockSpec((H, OUT), lambda i: (0, 0)),
                  pl.BlockSpec((1, OUT), lambda i: (0, 0))],
        out_specs=pl.BlockSpec((G, OUT), lambda i: (0, 0)),
        out_shape=jax.ShapeDtypeStruct((G, OUT), jnp.float32),
        scratch_shapes=[pltpu.VMEM((G, H), jnp.float32),
                        pltpu.VMEM((G, H), jnp.float32)],
    )(q, v, b1, W2, b2, batch3, Wf1, bf1, Wf2, bf2)


def kernel(x, edge_index, batch, W1a, b1a, W2a, b2a, W1b, b1b, W2b, b2b,
           Wf1, bf1, Wf2, bf2):
    xp = jnp.pad(x, ((0, NPAD - N), (0, 0)))
    src = edge_index[0]
    dst = edge_index[1]
    fill = jnp.full((EPAD - E,), N, jnp.int32)
    nf = 16 * CF * CHUNK
    srcp = jnp.concatenate([src, fill])
    dstp = jnp.concatenate([dst, fill])
    src_f = srcp[:nf].reshape(16, CF, CHUNK)
    dst_f = dstp[:nf].reshape(16, CF, CHUNK)
    src_s = srcp[nf:].reshape(16, CS, CHUNK)
    dst_s = dstp[nf:].reshape(16, CS, CHUNK)
    batch3 = jnp.concatenate(
        [batch, jnp.full((NPAD - N,), G, jnp.int32)]).reshape(NB, 1, BLK)

    zrows = jnp.zeros((ROWS_PER_TILE, H), jnp.float32)
    u = _k1(xp, W1a)
    p = _segment_sum_sc(u, src_f, dst_f, src_s, dst_s, zrows).reshape(2, NPAD, H)
    v = _k2(p, u, b1a.reshape(1, H), W2a, b2a.reshape(1, H), W1b)
    q = _segment_sum_sc(v, src_f, dst_f, src_s, dst_s, zrows).reshape(2, NPAD, H)
    out = _k3(q, v, b1b.reshape(1, H), W2b, b2b.reshape(1, H), batch3,
              Wf1, bf1.reshape(1, H), Wf2, bf2.reshape(1, OUT))
    return out


# speedup vs baseline: 2.0936x; 1.0196x over previous
import functools

import jax
import jax.numpy as jnp
from jax import lax
from jax.experimental import pallas as pl
from jax.experimental.pallas import tpu as pltpu
from jax.experimental.pallas import tpu_sc as plsc

N = 10000
NPAD = 10240
D = 128
H = 64
G = 64
OUT = 6
E = 320000
NW = 32
CHUNK = 80
CPT = 128
EPAD = NW * CPT * CHUNK
ROWS_PER_TILE = NPAD // 16
BLK = 1024
NB = NPAD // BLK


def _segment_sum_sc(u, src3, dst3, zrows):
    mesh = plsc.VectorSubcoreMesh(core_axis_name="c", subcore_axis_name="s")
    NBUF = 4

    @functools.partial(
        pl.kernel,
        out_type=jax.ShapeDtypeStruct((2 * NPAD, H), jnp.float32),
        mesh=mesh,
        compiler_params=pltpu.CompilerParams(use_tc_tiling_on_sc=False),
        scratch_types=[
            pltpu.VMEM((CPT, CHUNK), jnp.int32),
            pltpu.VMEM((CPT, CHUNK), jnp.int32),
            pltpu.VMEM((CHUNK, H), jnp.float32),
            pltpu.VMEM((CHUNK, H), jnp.float32),
            pltpu.VMEM((CHUNK, H), jnp.float32),
            pltpu.VMEM((CHUNK, H), jnp.float32),
            pltpu.VMEM_SHARED((NPAD, H), jnp.float32),
            pltpu.VMEM_SHARED((NPAD, H), jnp.float32),
            pltpu.SemaphoreType.DMA,
            pltpu.SemaphoreType.DMA,
            pltpu.SemaphoreType.DMA,
            pltpu.SemaphoreType.DMA,
        ],
    )
    def body(u_hbm, src_hbm, dst_hbm, z_hbm, out_hbm,
             src_v, dst_v, r0, r1, r2, r3, acc, u_s,
             g0, g1, g2, g3):
        rows = (r0, r1, r2, r3)
        gsem = (g0, g1, g2, g3)
        cid = lax.axis_index("c")
        sid = lax.axis_index("s")
        wid = cid * 16 + sid
        pltpu.sync_copy(src_hbm.at[wid], src_v)
        pltpu.sync_copy(dst_hbm.at[wid], dst_v)
        pltpu.sync_copy(
            u_hbm.at[pl.ds(sid * ROWS_PER_TILE, ROWS_PER_TILE)],
            u_s.at[pl.ds(sid * ROWS_PER_TILE, ROWS_PER_TILE)])
        pltpu.sync_copy(z_hbm, acc.at[pl.ds(sid * ROWS_PER_TILE, ROWS_PER_TILE)])
        plsc.subcore_barrier()

        for b in range(NBUF):
            pltpu.async_copy(u_s.at[src_v.at[b]], rows[b], gsem[b])

        def step(i, carry):
            j = NBUF * i
            for b in range(NBUF):
                pltpu.make_async_copy(u_s.at[src_v.at[j + b]], rows[b],
                                      gsem[b]).wait()
                pltpu.sync_copy(rows[b], acc.at[dst_v.at[j + b]], add=True)

                @pl.when(j + NBUF + b < CPT)
                def _():
                    pltpu.async_copy(u_s.at[src_v.at[j + NBUF + b]], rows[b],
                                     gsem[b])
            return carry

        lax.fori_loop(0, CPT // NBUF, step, 0)
        plsc.subcore_barrier()
        pltpu.sync_copy(
            acc.at[pl.ds(sid * ROWS_PER_TILE, ROWS_PER_TILE)],
            out_hbm.at[pl.ds(cid * NPAD + sid * ROWS_PER_TILE, ROWS_PER_TILE)])

    return body(u, src3, dst3, zrows)


def _k1(xp, W1a):
    def body(x_ref, w_ref, o_ref):
        o_ref[...] = jnp.dot(x_ref[...], w_ref[...],
                             preferred_element_type=jnp.float32)

    return pl.pallas_call(
        body,
        grid=(NB,),
        in_specs=[pl.BlockSpec((BLK, D), lambda i: (i, 0)),
                  pl.BlockSpec((D, H), lambda i: (0, 0))],
        out_specs=pl.BlockSpec((BLK, H), lambda i: (i, 0)),
        out_shape=jax.ShapeDtypeStruct((NPAD, H), jnp.float32),
    )(xp, W1a)


def _k2(p, u, b1, W2, b2, Wn):
    def body(p_ref, u_ref, b1_ref, w2_ref, b2_ref, wn_ref, o_ref):
        t = jnp.maximum(p_ref[0] + p_ref[1] + u_ref[...] + b1_ref[...], 0.0)
        h = jnp.maximum(
            jnp.dot(t, w2_ref[...], preferred_element_type=jnp.float32)
            + b2_ref[...], 0.0)
        o_ref[...] = jnp.dot(h, wn_ref[...], preferred_element_type=jnp.float32)

    return pl.pallas_call(
        body,
        grid=(NB,),
        in_specs=[pl.BlockSpec((2, BLK, H), lambda i: (0, i, 0)),
                  pl.BlockSpec((BLK, H), lambda i: (i, 0)),
                  pl.BlockSpec((1, H), lambda i: (0, 0)),
                  pl.BlockSpec((H, H), lambda i: (0, 0)),
                  pl.BlockSpec((1, H), lambda i: (0, 0)),
                  pl.BlockSpec((H, H), lambda i: (0, 0))],
        out_specs=pl.BlockSpec((BLK, H), lambda i: (i, 0)),
        out_shape=jax.ShapeDtypeStruct((NPAD, H), jnp.float32),
    )(p, u, b1, W2, b2, Wn)


def _k3(q, v, b1, W2, b2, batch3, Wf1, bf1, Wf2, bf2):
    def body(q_ref, v_ref, b1_ref, w2_ref, b2_ref, bt_ref, wf1_ref, bf1_ref,
             wf2_ref, bf2_ref, o_ref, sum_ref, cnt_ref):
        i = pl.program_id(0)
        t = jnp.maximum(q_ref[0] + q_ref[1] + v_ref[...] + b1_ref[...], 0.0)
        h2 = jnp.maximum(
            jnp.dot(t, w2_ref[...], preferred_element_type=jnp.float32)
            + b2_ref[...], 0.0)
        bv = bt_ref[0]
        oh = (lax.broadcasted_iota(jnp.int32, (G, BLK), 0)
              == jnp.broadcast_to(bv, (G, BLK))).astype(jnp.float32)
        s = lax.dot_general(oh, h2, (((1,), (0,)), ((), ())),
                            preferred_element_type=jnp.float32)
        c = lax.dot_general(oh, jnp.ones((BLK, H), jnp.float32),
                            (((1,), (0,)), ((), ())),
                            preferred_element_type=jnp.float32)

        @pl.when(i == 0)
        def _():
            sum_ref[...] = s
            cnt_ref[...] = c

        @pl.when(i > 0)
        def _():
            sum_ref[...] += s
            cnt_ref[...] += c

        @pl.when(i == NB - 1)
        def _():
            pooled = sum_ref[...] / jnp.maximum(cnt_ref[...], 1.0)
            o1 = jnp.maximum(
                jnp.dot(pooled, wf1_ref[...], preferred_element_type=jnp.float32)
                + bf1_ref[...], 0.0)
            logits = (jnp.dot(o1, wf2_ref[...], preferred_element_type=jnp.float32)
                      + bf2_ref[...])
            o_ref[...] = 1.0 / (1.0 + jnp.exp(-logits))

    return pl.pallas_call(
        body,
        grid=(NB,),
        in_specs=[pl.BlockSpec((2, BLK, H), lambda i: (0, i, 0)),
                  pl.BlockSpec((BLK, H), lambda i: (i, 0)),
                  pl.BlockSpec((1, H), lambda i: (0, 0)),
                  pl.BlockSpec((H, H), lambda i: (0, 0)),
                  pl.BlockSpec((1, H), lambda i: (0, 0)),
                  pl.BlockSpec((1, 1, BLK), lambda i: (i, 0, 0)),
                  pl.BlockSpec((H, H), lambda i: (0, 0)),
                  pl.BlockSpec((1, H), lambda i: (0, 0)),
                  pl.BlockSpec((H, OUT), lambda i: (0, 0)),
                  pl.BlockSpec((1, OUT), lambda i: (0, 0))],
        out_specs=pl.BlockSpec((G, OUT), lambda i: (0, 0)),
        out_shape=jax.ShapeDtypeStruct((G, OUT), jnp.float32),
        scratch_shapes=[pltpu.VMEM((G, H), jnp.float32),
                        pltpu.VMEM((G, H), jnp.float32)],
    )(q, v, b1, W2, b2, batch3, Wf1, bf1, Wf2, bf2)


def kernel(x, edge_index, batch, W1a, b1a, W2a, b2a, W1b, b1b, W2b, b2b,
           Wf1, bf1, Wf2, bf2):
    xp = jnp.pad(x, ((0, NPAD - N), (0, 0)))
    src = edge_index[0]
    dst = edge_index[1]
    fill = jnp.full((EPAD - E,), N, jnp.int32)
    src3 = jnp.concatenate([src, fill]).reshape(NW, CPT, CHUNK)
    dst3 = jnp.concatenate([dst, fill]).reshape(NW, CPT, CHUNK)
    batch3 = jnp.concatenate(
        [batch, jnp.full((NPAD - N,), G, jnp.int32)]).reshape(NB, 1, BLK)

    zrows = jnp.zeros((ROWS_PER_TILE, H), jnp.float32)
    u = _k1(xp, W1a)
    p = _segment_sum_sc(u, src3, dst3, zrows).reshape(2, NPAD, H)
    v = _k2(p, u, b1a.reshape(1, H), W2a, b2a.reshape(1, H), W1b)
    q = _segment_sum_sc(v, src3, dst3, zrows).reshape(2, NPAD, H)
    out = _k3(q, v, b1b.reshape(1, H), W2b, b2b.reshape(1, H), batch3,
              Wf1, bf1.reshape(1, H), Wf2, bf2.reshape(1, OUT))
    return out


# speedup vs baseline: 2.1528x; 1.0283x over previous
import functools

import jax
import jax.numpy as jnp
from jax import lax
from jax.experimental import pallas as pl
from jax.experimental.pallas import tpu as pltpu
from jax.experimental.pallas import tpu_sc as plsc

N = 10000
NPAD = 10240
D = 128
H = 64
G = 64
OUT = 6
E = 320000
NW = 32
CHUNK = 80
CPT = 125
ROWS_PER_TILE = NPAD // 16
BLK = 1024
NB = NPAD // BLK


def _segment_sum_sc(u, src3, dst3, zrows):
    mesh = plsc.VectorSubcoreMesh(core_axis_name="c", subcore_axis_name="s")
    NBUF = 5

    @functools.partial(
        pl.kernel,
        out_type=jax.ShapeDtypeStruct((2, NPAD, H), jnp.float32),
        mesh=mesh,
        compiler_params=pltpu.CompilerParams(use_tc_tiling_on_sc=False),
        scratch_types=[
            pltpu.VMEM((CPT, CHUNK), jnp.int32),
            pltpu.VMEM((CPT, CHUNK), jnp.int32),
            pltpu.VMEM((CHUNK, H), jnp.float32),
            pltpu.VMEM((CHUNK, H), jnp.float32),
            pltpu.VMEM((CHUNK, H), jnp.float32),
            pltpu.VMEM((CHUNK, H), jnp.float32),
            pltpu.VMEM((CHUNK, H), jnp.float32),
            pltpu.VMEM_SHARED((NPAD, H), jnp.float32),
            pltpu.VMEM_SHARED((NPAD, H), jnp.float32),
            pltpu.SemaphoreType.DMA,
            pltpu.SemaphoreType.DMA,
            pltpu.SemaphoreType.DMA,
            pltpu.SemaphoreType.DMA,
            pltpu.SemaphoreType.DMA,
        ],
    )
    def body(u_hbm, src_hbm, dst_hbm, z_hbm, out_hbm,
             src_v, dst_v, r0, r1, r2, r3, r4, acc, u_s,
             g0, g1, g2, g3, g4):
        rows = (r0, r1, r2, r3, r4)
        gsem = (g0, g1, g2, g3, g4)
        cid = lax.axis_index("c")
        sid = lax.axis_index("s")
        wid = cid * 16 + sid
        pltpu.sync_copy(src_hbm.at[wid], src_v)
        pltpu.sync_copy(dst_hbm.at[wid], dst_v)
        pltpu.sync_copy(
            u_hbm.at[pl.ds(sid * ROWS_PER_TILE, ROWS_PER_TILE)],
            u_s.at[pl.ds(sid * ROWS_PER_TILE, ROWS_PER_TILE)])
        pltpu.sync_copy(z_hbm, acc.at[pl.ds(sid * ROWS_PER_TILE, ROWS_PER_TILE)])
        plsc.subcore_barrier()

        for b in range(NBUF):
            pltpu.async_copy(u_s.at[src_v.at[b]], rows[b], gsem[b])

        def step(i, carry):
            j = NBUF * i
            for b in range(NBUF):
                pltpu.make_async_copy(u_s.at[src_v.at[j + b]], rows[b],
                                      gsem[b]).wait()
                pltpu.sync_copy(rows[b], acc.at[dst_v.at[j + b]], add=True)

                @pl.when(j + NBUF + b < CPT)
                def _():
                    pltpu.async_copy(u_s.at[src_v.at[j + NBUF + b]], rows[b],
                                     gsem[b])
            return carry

        lax.fori_loop(0, CPT // NBUF, step, 0)
        plsc.subcore_barrier()
        pltpu.sync_copy(
            acc.at[pl.ds(sid * ROWS_PER_TILE, ROWS_PER_TILE)],
            out_hbm.at[cid].at[pl.ds(sid * ROWS_PER_TILE, ROWS_PER_TILE)])

    return body(u, src3, dst3, zrows)


def _k1(xp, W1a):
    def body(x_ref, w_ref, o_ref):
        o_ref[...] = jnp.dot(x_ref[...], w_ref[...],
                             preferred_element_type=jnp.float32)

    return pl.pallas_call(
        body,
        grid=(NB,),
        in_specs=[pl.BlockSpec((BLK, D), lambda i: (i, 0)),
                  pl.BlockSpec((D, H), lambda i: (0, 0))],
        out_specs=pl.BlockSpec((BLK, H), lambda i: (i, 0)),
        out_shape=jax.ShapeDtypeStruct((NPAD, H), jnp.float32),
    )(xp, W1a)


def _k2(p, u, b1, W2, b2, Wn):
    def body(p_ref, u_ref, b1_ref, w2_ref, b2_ref, wn_ref, o_ref):
        t = jnp.maximum(p_ref[0] + p_ref[1] + u_ref[...] + b1_ref[...], 0.0)
        h = jnp.maximum(
            jnp.dot(t, w2_ref[...], preferred_element_type=jnp.float32)
            + b2_ref[...], 0.0)
        o_ref[...] = jnp.dot(h, wn_ref[...], preferred_element_type=jnp.float32)

    return pl.pallas_call(
        body,
        grid=(NB,),
        in_specs=[pl.BlockSpec((2, BLK, H), lambda i: (0, i, 0)),
                  pl.BlockSpec((BLK, H), lambda i: (i, 0)),
                  pl.BlockSpec((1, H), lambda i: (0, 0)),
                  pl.BlockSpec((H, H), lambda i: (0, 0)),
                  pl.BlockSpec((1, H), lambda i: (0, 0)),
                  pl.BlockSpec((H, H), lambda i: (0, 0))],
        out_specs=pl.BlockSpec((BLK, H), lambda i: (i, 0)),
        out_shape=jax.ShapeDtypeStruct((NPAD, H), jnp.float32),
    )(p, u, b1, W2, b2, Wn)


def _k3(q, v, b1, W2, b2, batch3, Wf1, bf1, Wf2, bf2):
    def body(q_ref, v_ref, b1_ref, w2_ref, b2_ref, bt_ref, wf1_ref, bf1_ref,
             wf2_ref, bf2_ref, o_ref, sum_ref, cnt_ref):
        i = pl.program_id(0)
        t = jnp.maximum(q_ref[0] + q_ref[1] + v_ref[...] + b1_ref[...], 0.0)
        h2 = jnp.maximum(
            jnp.dot(t, w2_ref[...], preferred_element_type=jnp.float32)
            + b2_ref[...], 0.0)
        bv = bt_ref[0]
        oh = (lax.broadcasted_iota(jnp.int32, (G, BLK), 0)
              == jnp.broadcast_to(bv, (G, BLK))).astype(jnp.float32)
        s = lax.dot_general(oh, h2, (((1,), (0,)), ((), ())),
                            preferred_element_type=jnp.float32)
        c = lax.dot_general(oh, jnp.ones((BLK, H), jnp.float32),
                            (((1,), (0,)), ((), ())),
                            preferred_element_type=jnp.float32)

        @pl.when(i == 0)
        def _():
            sum_ref[...] = s
            cnt_ref[...] = c

        @pl.when(i > 0)
        def _():
            sum_ref[...] += s
            cnt_ref[...] += c

        @pl.when(i == NB - 1)
        def _():
            pooled = sum_ref[...] / jnp.maximum(cnt_ref[...], 1.0)
            o1 = jnp.maximum(
                jnp.dot(pooled, wf1_ref[...], preferred_element_type=jnp.float32)
                + bf1_ref[...], 0.0)
            logits = (jnp.dot(o1, wf2_ref[...], preferred_element_type=jnp.float32)
                      + bf2_ref[...])
            o_ref[...] = 1.0 / (1.0 + jnp.exp(-logits))

    return pl.pallas_call(
        body,
        grid=(NB,),
        in_specs=[pl.BlockSpec((2, BLK, H), lambda i: (0, i, 0)),
                  pl.BlockSpec((BLK, H), lambda i: (i, 0)),
                  pl.BlockSpec((1, H), lambda i: (0, 0)),
                  pl.BlockSpec((H, H), lambda i: (0, 0)),
                  pl.BlockSpec((1, H), lambda i: (0, 0)),
                  pl.BlockSpec((1, 1, BLK), lambda i: (i, 0, 0)),
                  pl.BlockSpec((H, H), lambda i: (0, 0)),
                  pl.BlockSpec((1, H), lambda i: (0, 0)),
                  pl.BlockSpec((H, OUT), lambda i: (0, 0)),
                  pl.BlockSpec((1, OUT), lambda i: (0, 0))],
        out_specs=pl.BlockSpec((G, OUT), lambda i: (0, 0)),
        out_shape=jax.ShapeDtypeStruct((G, OUT), jnp.float32),
        scratch_shapes=[pltpu.VMEM((G, H), jnp.float32),
                        pltpu.VMEM((G, H), jnp.float32)],
    )(q, v, b1, W2, b2, batch3, Wf1, bf1, Wf2, bf2)


def kernel(x, edge_index, batch, W1a, b1a, W2a, b2a, W1b, b1b, W2b, b2b,
           Wf1, bf1, Wf2, bf2):
    xp = jnp.pad(x, ((0, NPAD - N), (0, 0)))
    src3 = edge_index[0].reshape(NW, CPT, CHUNK)
    dst3 = edge_index[1].reshape(NW, CPT, CHUNK)
    batch3 = jnp.concatenate(
        [batch, jnp.full((NPAD - N,), G, jnp.int32)]).reshape(NB, 1, BLK)

    zrows = jnp.zeros((ROWS_PER_TILE, H), jnp.float32)
    u = _k1(xp, W1a)
    p = _segment_sum_sc(u, src3, dst3, zrows)
    v = _k2(p, u, b1a.reshape(1, H), W2a, b2a.reshape(1, H), W1b)
    q = _segment_sum_sc(v, src3, dst3, zrows)
    out = _k3(q, v, b1b.reshape(1, H), W2b, b2b.reshape(1, H), batch3,
              Wf1, bf1.reshape(1, H), Wf2, bf2.reshape(1, OUT))
    return out
